# baseline (speedup 1.0000x reference)
import functools

import jax
import jax.numpy as jnp
from jax import lax
from jax.experimental import pallas as pl
from jax.experimental.pallas import tpu as pltpu
from jax.experimental.pallas import tpu_sc as plsc

_E, _K, _D, _DFF, _T = 16, 2, 1024, 1536, 8192
_B = 128
_MAX_NB = _T * _K // _B + _E
_PAD = _MAX_NB * _B
_TB = 512
_NTB = _T // _TB

_NC, _NS = 2, 16
_NW = _NC * _NS
_CH = 64


def _top2(logits):
    iota = lax.broadcasted_iota(jnp.int32, logits.shape, 1)
    m1 = jnp.max(logits, axis=1, keepdims=True)
    i1 = jnp.min(jnp.where(logits == m1, iota, _E), axis=1, keepdims=True)
    masked = jnp.where(iota == i1, -jnp.inf, logits)
    m2 = jnp.max(masked, axis=1, keepdims=True)
    i2 = jnp.min(jnp.where(masked == m2, iota, _E), axis=1, keepdims=True)
    return m1, m2, i1, i2


def _router_body(x_ref, gw_ref, idx_ref, w_ref, rank_ref, cnt_ref, run_ref):
    @pl.when(pl.program_id(0) == 0)
    def _():
        run_ref[...] = jnp.zeros((1, _E), jnp.float32)

    logits = lax.dot_general(x_ref[...], gw_ref[...], (((1,), (1,)), ((), ())),
                             preferred_element_type=jnp.float32)
    m1, m2, i1, i2 = _top2(logits)
    idx_ref[...] = jnp.concatenate([i1, i2], axis=1)
    w1 = jax.nn.sigmoid(m1 - m2)
    w_ref[...] = jnp.concatenate([w1, 1.0 - w1], axis=1)

    iota_e = lax.broadcasted_iota(jnp.int32, (_TB, _E), 1)
    oh1 = (iota_e == i1).astype(jnp.float32)
    oh2 = (iota_e == i2).astype(jnp.float32)
    both = oh1 + oh2
    r_i = lax.broadcasted_iota(jnp.int32, (_TB, _TB), 0)
    c_i = lax.broadcasted_iota(jnp.int32, (_TB, _TB), 1)
    tril = (r_i > c_i).astype(jnp.float32)
    pref = lax.dot_general(tril, both, (((1,), (0,)), ((), ())),
                           preferred_element_type=jnp.float32)
    pref = pref + run_ref[...]
    rank1 = jnp.sum(pref * oh1, axis=1, keepdims=True)
    rank2 = jnp.sum((pref + oh1) * oh2, axis=1, keepdims=True)
    rank_ref[...] = jnp.concatenate([rank1, rank2], axis=1).astype(jnp.int32)
    newrun = run_ref[...] + jnp.sum(both, axis=0, keepdims=True)
    run_ref[...] = newrun
    cnt_ref[...] = newrun.astype(jnp.int32)[None]


_router = pl.pallas_call(
    _router_body,
    grid=(_NTB,),
    in_specs=[pl.BlockSpec((_TB, _D), lambda b: (b, 0)),
              pl.BlockSpec((_E, _D), lambda b: (0, 0))],
    out_specs=(pl.BlockSpec((_TB, 2), lambda b: (b, 0)),
               pl.BlockSpec((_TB, 2), lambda b: (b, 0)),
               pl.BlockSpec((_TB, 2), lambda b: (b, 0)),
               pl.BlockSpec((1, 1, _E), lambda b: (b, 0, 0))),
    out_shape=(jax.ShapeDtypeStruct((_T, 2), jnp.int32),
               jax.ShapeDtypeStruct((_T, 2), jnp.float32),
               jax.ShapeDtypeStruct((_T, 2), jnp.int32),
               jax.ShapeDtypeStruct((_NTB, 1, _E), jnp.int32)),
    scratch_shapes=[pltpu.VMEM((1, _E), jnp.float32)],
)


@functools.cache
def _make_gather(n_rows, dtype=jnp.float32):
    n_w = n_rows // _NW
    ch = 64
    n_chunks = n_w // ch
    assert n_rows % _NW == 0 and n_w % ch == 0
    mesh = plsc.VectorSubcoreMesh(core_axis_name="c", subcore_axis_name="s")

    @functools.partial(
        pl.kernel,
        out_type=jax.ShapeDtypeStruct((n_rows, _D), dtype),
        mesh=mesh,
        scratch_types=[pltpu.VMEM((n_w,), jnp.int32),
                       pltpu.VMEM((ch, _D), dtype),
                       pltpu.SemaphoreType.DMA],
    )
    def gather(src_hbm, idx_hbm, out_hbm, idx_v, rows_v, gs):
        wid = lax.axis_index("s") * _NC + lax.axis_index("c")
        base = wid * n_w
        pltpu.sync_copy(idx_hbm.at[pl.ds(base, n_w)], idx_v)

        def body(i, carry):
            pltpu.async_copy(
                src_hbm.at[idx_v.at[pl.ds(i * ch, ch)]], rows_v, gs).wait()
            pltpu.sync_copy(rows_v, out_hbm.at[pl.ds(base + i * ch, ch)])
            return carry

        lax.fori_loop(0, n_chunks, body, 0)

    return gather


_DCH = 32
_DNCH = _T // _NW // _DCH


@functools.cache
def _make_dispatch():
    mesh = plsc.VectorSubcoreMesh(core_axis_name="c", subcore_axis_name="s")
    n_w = _T // _NW

    @functools.partial(
        pl.kernel,
        out_type=jax.ShapeDtypeStruct((_PAD, _D), jnp.float32),
        mesh=mesh,
        scratch_types=[pltpu.VMEM((_DNCH, _DCH), jnp.int32),
                       pltpu.VMEM((_DNCH, _DCH), jnp.int32),
                       pltpu.VMEM((2, _DCH, _D), jnp.float32),
                       pltpu.SemaphoreType.DMA,
                       pltpu.SemaphoreType.DMA,
                       pltpu.SemaphoreType.DMA,
                       pltpu.SemaphoreType.DMA,
                       pltpu.SemaphoreType.DMA,
                       pltpu.SemaphoreType.DMA],
    )
    def dispatch(x_hbm, p1_hbm, p2_hbm, out_hbm, idx1_v, idx2_v, rows_v,
                 gs0, gs1, sa0, sa1, sb0, sb1):
        wid = lax.axis_index("s") * _NC + lax.axis_index("c")
        base = wid * n_w
        gsems, asems, bsems = (gs0, gs1), (sa0, sa1), (sb0, sb1)
        pltpu.sync_copy(p1_hbm.at[wid], idx1_v)
        pltpu.sync_copy(p2_hbm.at[wid], idx2_v)

        g = [None] * _DNCH
        s1 = [None] * _DNCH
        s2 = [None] * _DNCH
        g[0] = pltpu.async_copy(
            x_hbm.at[pl.ds(base, _DCH)], rows_v.at[0], gsems[0])
        for i in range(_DNCH):
            b = i % 2
            j = i + 1
            if j < _DNCH:
                if j >= 2:
                    s1[j - 2].wait()
                    s2[j - 2].wait()
                g[j] = pltpu.async_copy(
                    x_hbm.at[pl.ds(base + j * _DCH, _DCH)],
                    rows_v.at[j % 2], gsems[j % 2])
            g[i].wait()
            s1[i] = pltpu.async_copy(
                rows_v.at[b], out_hbm.at[idx1_v.at[i]], asems[b])
            s2[i] = pltpu.async_copy(
                rows_v.at[b], out_hbm.at[idx2_v.at[i]], bsems[b])
        if _DNCH > 1:
            s1[_DNCH - 2].wait()
            s2[_DNCH - 2].wait()
        s1[_DNCH - 1].wait()
        s2[_DNCH - 1].wait()

    return dispatch


def _ffn_body(be_ref, xs_ref, wg_ref, wu_ref, wd_ref, y_ref):
    del be_ref
    xb = xs_ref[...]
    g = lax.dot_general(xb, wg_ref[0], (((1,), (1,)), ((), ())),
                        preferred_element_type=jnp.float32)
    u = lax.dot_general(xb, wu_ref[0], (((1,), (1,)), ((), ())),
                        preferred_element_type=jnp.float32)
    h = g * jax.nn.sigmoid(g) * u
    y_ref[...] = lax.dot_general(h, wd_ref[0], (((1,), (1,)), ((), ())),
                                 preferred_element_type=jnp.float32)


_ffn = pl.pallas_call(
    _ffn_body,
    grid_spec=pltpu.PrefetchScalarGridSpec(
        num_scalar_prefetch=1,
        grid=(_MAX_NB,),
        in_specs=[pl.BlockSpec((_B, _D), lambda b, be: (b, 0)),
                  pl.BlockSpec((1, _DFF, _D), lambda b, be: (be[b], 0, 0)),
                  pl.BlockSpec((1, _DFF, _D), lambda b, be: (be[b], 0, 0)),
                  pl.BlockSpec((1, _D, _DFF), lambda b, be: (be[b], 0, 0))],
        out_specs=pl.BlockSpec((_B, _D), lambda b, be: (b, 0)),
    ),
    out_shape=jax.ShapeDtypeStruct((_PAD, _D), jnp.float32),
)


def _shared_body(x_ref, wgs_ref, wus_ref, wds_ref, o_ref):
    xb = x_ref[...]
    g = lax.dot_general(xb, wgs_ref[...], (((1,), (1,)), ((), ())),
                        preferred_element_type=jnp.float32)
    u = lax.dot_general(xb, wus_ref[...], (((1,), (1,)), ((), ())),
                        preferred_element_type=jnp.float32)
    h = g * jax.nn.sigmoid(g) * u
    o_ref[...] = lax.dot_general(h, wds_ref[...], (((1,), (1,)), ((), ())),
                                 preferred_element_type=jnp.float32)


_shared = pl.pallas_call(
    _shared_body,
    grid=(_NTB,),
    in_specs=[pl.BlockSpec((_TB, _D), lambda b: (b, 0)),
              pl.BlockSpec((_DFF, _D), lambda b: (0, 0)),
              pl.BlockSpec((_DFF, _D), lambda b: (0, 0)),
              pl.BlockSpec((_D, _DFF), lambda b: (0, 0))],
    out_specs=pl.BlockSpec((_TB, _D), lambda b: (b, 0)),
    out_shape=jax.ShapeDtypeStruct((_T, _D), jnp.float32),
)


def _combine_body(sh_ref, w_ref, g1_ref, g2_ref, o_ref):
    w = w_ref[...]
    o_ref[...] = (sh_ref[...] + w[:, 0:1] * g1_ref[...]
                  + w[:, 1:2] * g2_ref[...])


_combine = pl.pallas_call(
    _combine_body,
    grid=(_NTB,),
    in_specs=[pl.BlockSpec((_TB, _D), lambda b: (b, 0)),
              pl.BlockSpec((_TB, 2), lambda b: (b, 0)),
              pl.BlockSpec((_TB, _D), lambda b: (b, 0)),
              pl.BlockSpec((_TB, _D), lambda b: (b + _NTB, 0))],
    out_specs=pl.BlockSpec((_TB, _D), lambda b: (b, 0)),
    out_shape=jax.ShapeDtypeStruct((_T, _D), jnp.float32),
)


def kernel(x, gate_w, Wg, Wu, Wd, Wgs, Wus, Wds):
    idx, w, rank, cnt = _router(x, gate_w)
    sh = _shared(x, Wgs, Wus, Wds)

    counts = cnt[-1, 0]
    eflat = idx.reshape(-1)
    padded = ((counts + _B - 1) // _B) * _B
    ends = jnp.cumsum(padded)
    starts = ends - padded
    dest = starts[eflat] + rank.reshape(-1)
    block_expert = jnp.minimum(
        jnp.searchsorted(ends, jnp.arange(_MAX_NB) * _B, side="right"),
        _E - 1).astype(jnp.int32)

    dre = dest.reshape(_T, _K)
    p1 = dre[:, 0].reshape(_NW, _DNCH, _DCH)
    p2 = dre[:, 1].reshape(_NW, _DNCH, _DCH)
    xs = _make_dispatch()(x, p1, p2)
    y = _ffn(block_expert, xs, Wg, Wu, Wd)

    pcat = jnp.concatenate([dre[:, 0], dre[:, 1]])
    gcat = _make_gather(_T * _K)(y, pcat)

    return _combine(sh, w, gcat, gcat)


# speedup vs baseline: 1.4805x; 1.4805x over previous
import functools

import jax
import jax.numpy as jnp
from jax import lax
from jax.experimental import pallas as pl
from jax.experimental.pallas import tpu as pltpu
from jax.experimental.pallas import tpu_sc as plsc

_E, _K, _D, _DFF, _T = 16, 2, 1024, 1536, 8192
_B = 512
_MAX_NB = _T * _K // _B + _E
_PAD = _MAX_NB * _B
_TB = 512
_NTB = _T // _TB

_NC, _NS = 2, 16
_NW = _NC * _NS
_CH = 64


def _top2(logits):
    iota = lax.broadcasted_iota(jnp.int32, logits.shape, 1)
    m1 = jnp.max(logits, axis=1, keepdims=True)
    i1 = jnp.min(jnp.where(logits == m1, iota, _E), axis=1, keepdims=True)
    masked = jnp.where(iota == i1, -jnp.inf, logits)
    m2 = jnp.max(masked, axis=1, keepdims=True)
    i2 = jnp.min(jnp.where(masked == m2, iota, _E), axis=1, keepdims=True)
    return m1, m2, i1, i2


def _router_body(x_ref, gw_ref, idx_ref, w_ref, rank_ref, cnt_ref, run_ref):
    @pl.when(pl.program_id(0) == 0)
    def _():
        run_ref[...] = jnp.zeros((1, _E), jnp.float32)

    logits = lax.dot_general(x_ref[...], gw_ref[...], (((1,), (1,)), ((), ())),
                             preferred_element_type=jnp.float32)
    m1, m2, i1, i2 = _top2(logits)
    idx_ref[...] = jnp.concatenate([i1, i2], axis=1)
    w1 = jax.nn.sigmoid(m1 - m2)
    w_ref[...] = jnp.concatenate([w1, 1.0 - w1], axis=1)

    iota_e = lax.broadcasted_iota(jnp.int32, (_TB, _E), 1)
    oh1 = (iota_e == i1).astype(jnp.float32)
    oh2 = (iota_e == i2).astype(jnp.float32)
    both = oh1 + oh2
    r_i = lax.broadcasted_iota(jnp.int32, (_TB, _TB), 0)
    c_i = lax.broadcasted_iota(jnp.int32, (_TB, _TB), 1)
    tril = (r_i > c_i).astype(jnp.float32)
    pref = lax.dot_general(tril, both, (((1,), (0,)), ((), ())),
                           preferred_element_type=jnp.float32)
    pref = pref + run_ref[...]
    rank1 = jnp.sum(pref * oh1, axis=1, keepdims=True)
    rank2 = jnp.sum((pref + oh1) * oh2, axis=1, keepdims=True)
    rank_ref[...] = jnp.concatenate([rank1, rank2], axis=1).astype(jnp.int32)
    newrun = run_ref[...] + jnp.sum(both, axis=0, keepdims=True)
    run_ref[...] = newrun
    cnt_ref[...] = newrun.astype(jnp.int32)[None]


_router = pl.pallas_call(
    _router_body,
    grid=(_NTB,),
    in_specs=[pl.BlockSpec((_TB, _D), lambda b: (b, 0)),
              pl.BlockSpec((_E, _D), lambda b: (0, 0))],
    out_specs=(pl.BlockSpec((_TB, 2), lambda b: (b, 0)),
               pl.BlockSpec((_TB, 2), lambda b: (b, 0)),
               pl.BlockSpec((_TB, 2), lambda b: (b, 0)),
               pl.BlockSpec((1, 1, _E), lambda b: (b, 0, 0))),
    out_shape=(jax.ShapeDtypeStruct((_T, 2), jnp.int32),
               jax.ShapeDtypeStruct((_T, 2), jnp.float32),
               jax.ShapeDtypeStruct((_T, 2), jnp.int32),
               jax.ShapeDtypeStruct((_NTB, 1, _E), jnp.int32)),
    scratch_shapes=[pltpu.VMEM((1, _E), jnp.float32)],
)


@functools.cache
def _make_gather(n_rows, dtype=jnp.float32):
    n_w = n_rows // _NW
    ch = 64
    n_chunks = n_w // ch
    assert n_rows % _NW == 0 and n_w % ch == 0
    mesh = plsc.VectorSubcoreMesh(core_axis_name="c", subcore_axis_name="s")

    @functools.partial(
        pl.kernel,
        out_type=jax.ShapeDtypeStruct((n_rows, _D), dtype),
        mesh=mesh,
        scratch_types=[pltpu.VMEM((n_w,), jnp.int32),
                       pltpu.VMEM((ch, _D), dtype),
                       pltpu.SemaphoreType.DMA],
    )
    def gather(src_hbm, idx_hbm, out_hbm, idx_v, rows_v, gs):
        wid = lax.axis_index("s") * _NC + lax.axis_index("c")
        base = wid * n_w
        pltpu.sync_copy(idx_hbm.at[pl.ds(base, n_w)], idx_v)

        def body(i, carry):
            pltpu.async_copy(
                src_hbm.at[idx_v.at[pl.ds(i * ch, ch)]], rows_v, gs).wait()
            pltpu.sync_copy(rows_v, out_hbm.at[pl.ds(base + i * ch, ch)])
            return carry

        lax.fori_loop(0, n_chunks, body, 0)

    return gather


_DCH = 32
_DNCH = _T // _NW // _DCH


@functools.cache
def _make_dispatch():
    mesh = plsc.VectorSubcoreMesh(core_axis_name="c", subcore_axis_name="s")
    n_w = _T // _NW

    @functools.partial(
        pl.kernel,
        out_type=jax.ShapeDtypeStruct((_PAD, _D), jnp.float32),
        mesh=mesh,
        scratch_types=[pltpu.VMEM((_DNCH, _DCH), jnp.int32),
                       pltpu.VMEM((_DNCH, _DCH), jnp.int32),
                       pltpu.VMEM((2, _DCH, _D), jnp.float32),
                       pltpu.SemaphoreType.DMA,
                       pltpu.SemaphoreType.DMA,
                       pltpu.SemaphoreType.DMA,
                       pltpu.SemaphoreType.DMA,
                       pltpu.SemaphoreType.DMA,
                       pltpu.SemaphoreType.DMA],
    )
    def dispatch(x_hbm, p1_hbm, p2_hbm, out_hbm, idx1_v, idx2_v, rows_v,
                 gs0, gs1, sa0, sa1, sb0, sb1):
        wid = lax.axis_index("s") * _NC + lax.axis_index("c")
        base = wid * n_w
        gsems, asems, bsems = (gs0, gs1), (sa0, sa1), (sb0, sb1)
        pltpu.sync_copy(p1_hbm.at[wid], idx1_v)
        pltpu.sync_copy(p2_hbm.at[wid], idx2_v)

        g = [None] * _DNCH
        s1 = [None] * _DNCH
        s2 = [None] * _DNCH
        g[0] = pltpu.async_copy(
            x_hbm.at[pl.ds(base, _DCH)], rows_v.at[0], gsems[0])
        for i in range(_DNCH):
            b = i % 2
            j = i + 1
            if j < _DNCH:
                if j >= 2:
                    s1[j - 2].wait()
                    s2[j - 2].wait()
                g[j] = pltpu.async_copy(
                    x_hbm.at[pl.ds(base + j * _DCH, _DCH)],
                    rows_v.at[j % 2], gsems[j % 2])
            g[i].wait()
            s1[i] = pltpu.async_copy(
                rows_v.at[b], out_hbm.at[idx1_v.at[i]], asems[b])
            s2[i] = pltpu.async_copy(
                rows_v.at[b], out_hbm.at[idx2_v.at[i]], bsems[b])
        if _DNCH > 1:
            s1[_DNCH - 2].wait()
            s2[_DNCH - 2].wait()
        s1[_DNCH - 1].wait()
        s2[_DNCH - 1].wait()

    return dispatch


def _ffn_body(be_ref, xs_ref, wg_ref, wu_ref, wd_ref, y_ref):
    del be_ref
    xb = xs_ref[...]
    g = lax.dot_general(xb, wg_ref[0], (((1,), (1,)), ((), ())),
                        preferred_element_type=jnp.float32)
    u = lax.dot_general(xb, wu_ref[0], (((1,), (1,)), ((), ())),
                        preferred_element_type=jnp.float32)
    h = g * jax.nn.sigmoid(g) * u
    y_ref[...] = lax.dot_general(h, wd_ref[0], (((1,), (1,)), ((), ())),
                                 preferred_element_type=jnp.float32)


_ffn = pl.pallas_call(
    _ffn_body,
    grid_spec=pltpu.PrefetchScalarGridSpec(
        num_scalar_prefetch=1,
        grid=(_MAX_NB,),
        in_specs=[pl.BlockSpec((_B, _D), lambda b, be: (b, 0)),
                  pl.BlockSpec((1, _DFF, _D), lambda b, be: (be[b], 0, 0)),
                  pl.BlockSpec((1, _DFF, _D), lambda b, be: (be[b], 0, 0)),
                  pl.BlockSpec((1, _D, _DFF), lambda b, be: (be[b], 0, 0))],
        out_specs=pl.BlockSpec((_B, _D), lambda b, be: (b, 0)),
    ),
    out_shape=jax.ShapeDtypeStruct((_PAD, _D), jnp.float32),
)


def _shared_body(x_ref, wgs_ref, wus_ref, wds_ref, o_ref):
    xb = x_ref[...]
    g = lax.dot_general(xb, wgs_ref[...], (((1,), (1,)), ((), ())),
                        preferred_element_type=jnp.float32)
    u = lax.dot_general(xb, wus_ref[...], (((1,), (1,)), ((), ())),
                        preferred_element_type=jnp.float32)
    h = g * jax.nn.sigmoid(g) * u
    o_ref[...] = lax.dot_general(h, wds_ref[...], (((1,), (1,)), ((), ())),
                                 preferred_element_type=jnp.float32)


_shared = pl.pallas_call(
    _shared_body,
    grid=(_NTB,),
    in_specs=[pl.BlockSpec((_TB, _D), lambda b: (b, 0)),
              pl.BlockSpec((_DFF, _D), lambda b: (0, 0)),
              pl.BlockSpec((_DFF, _D), lambda b: (0, 0)),
              pl.BlockSpec((_D, _DFF), lambda b: (0, 0))],
    out_specs=pl.BlockSpec((_TB, _D), lambda b: (b, 0)),
    out_shape=jax.ShapeDtypeStruct((_T, _D), jnp.float32),
)


def _combine_body(sh_ref, w_ref, g1_ref, g2_ref, o_ref):
    w = w_ref[...]
    o_ref[...] = (sh_ref[...] + w[:, 0:1] * g1_ref[...]
                  + w[:, 1:2] * g2_ref[...])


_combine = pl.pallas_call(
    _combine_body,
    grid=(_NTB,),
    in_specs=[pl.BlockSpec((_TB, _D), lambda b: (b, 0)),
              pl.BlockSpec((_TB, 2), lambda b: (b, 0)),
              pl.BlockSpec((_TB, _D), lambda b: (b, 0)),
              pl.BlockSpec((_TB, _D), lambda b: (b + _NTB, 0))],
    out_specs=pl.BlockSpec((_TB, _D), lambda b: (b, 0)),
    out_shape=jax.ShapeDtypeStruct((_T, _D), jnp.float32),
)


def kernel(x, gate_w, Wg, Wu, Wd, Wgs, Wus, Wds):
    idx, w, rank, cnt = _router(x, gate_w)
    sh = _shared(x, Wgs, Wus, Wds)

    counts = cnt[-1, 0]
    eflat = idx.reshape(-1)
    padded = ((counts + _B - 1) // _B) * _B
    ends = jnp.cumsum(padded)
    starts = ends - padded
    dest = starts[eflat] + rank.reshape(-1)
    block_expert = jnp.minimum(
        jnp.searchsorted(ends, jnp.arange(_MAX_NB) * _B, side="right"),
        _E - 1).astype(jnp.int32)

    dre = dest.reshape(_T, _K)
    p1 = dre[:, 0].reshape(_NW, _DNCH, _DCH)
    p2 = dre[:, 1].reshape(_NW, _DNCH, _DCH)
    xs = _make_dispatch()(x, p1, p2)
    y = _ffn(block_expert, xs, Wg, Wu, Wd)

    pcat = jnp.concatenate([dre[:, 0], dre[:, 1]])
    gcat = _make_gather(_T * _K)(y, pcat)

    return _combine(sh, w, gcat, gcat)
